# Initial kernel scaffold; baseline (speedup 1.0000x reference)
import jax
import jax.numpy as jnp
from jax.experimental import pallas as pl


def kernel(preds, plus_req, plus_var, minus_req, minus_var):
    raise NotImplementedError("write your pallas kernel here")


# speedup vs baseline: 1.2037x; 1.2037x over previous
import functools

import jax
import jax.numpy as jnp
from jax import lax
from jax.experimental import pallas as pl
from jax.experimental.pallas import tpu as pltpu
from jax.experimental.pallas import tpu_sc as plsc

_NUM_REQ = 512
_LITS = 4
_NC = 2
_NS = 16
_NW = _NC * _NS
_LANES = 16
_CHUNK = 16
_NGROUPS = _NUM_REQ // _LANES


def _body(preds_h, var_h, scale_h, off_h, out_h, var_v, scale_v, off_v,
          rows_v, acc_v, rows_per_w, num_vars):
    c = lax.axis_index("c")
    s = lax.axis_index("s")
    wid = s * _NC + c
    base = wid * rows_per_w
    nchunks = rows_per_w // _CHUNK

    pltpu.sync_copy(var_h, var_v)
    pltpu.sync_copy(scale_h, scale_v)
    pltpu.sync_copy(off_h, off_v)

    def g_body(g, accs):
        o = g * _LANES
        idx = [var_v[pl.ds(k * _NUM_REQ + o, _LANES)] for k in range(_LITS)]
        sc = [scale_v[pl.ds(k * _NUM_REQ + o, _LANES)] for k in range(_LITS)]
        of = [off_v[pl.ds(k * _NUM_REQ + o, _LANES)] for k in range(_LITS)]
        accs = list(accs)
        zero = jnp.zeros((_LANES,), jnp.float32)
        for r in range(_CHUNK):
            roff = jnp.full((_LANES,), r * num_vars, jnp.int32)
            vals = [of[k] + sc[k] * plsc.load_gather(rows_v, [roff + idx[k]])
                    for k in range(_LITS)]
            m = jnp.maximum(jnp.maximum(vals[0], vals[1]),
                            jnp.maximum(vals[2], vals[3]))
            m = jnp.maximum(m, zero)
            accs[r % 4] = accs[r % 4] + m
        return tuple(accs)

    def chunk_body(ci, accs):
        span = _CHUNK * num_vars
        pltpu.sync_copy(preds_h.at[pl.ds((base + ci * _CHUNK) * num_vars, span)],
                        rows_v)
        return lax.fori_loop(0, _NGROUPS, g_body, accs)

    z = jnp.zeros((_LANES,), jnp.float32)
    accs = lax.fori_loop(0, nchunks, chunk_body, (z, z, z, z))
    acc_v[...] = accs[0] + accs[1] + accs[2] + accs[3]
    pltpu.sync_copy(acc_v, out_h.at[wid])


def kernel(preds, plus_req, plus_var, minus_req, minus_var):
    batch, _ = preds.shape
    r_tot = _NUM_REQ
    n_plus = plus_req.shape[0]
    n_minus = minus_req.shape[0]

    cnt_p = jnp.zeros((r_tot,), jnp.int32).at[plus_req].add(1)
    offs_p = jnp.cumsum(cnt_p) - cnt_p
    rank_p = jnp.arange(n_plus, dtype=jnp.int32) - offs_p[plus_req]
    slot_p = rank_p * r_tot + plus_req

    cnt_m = jnp.zeros((r_tot,), jnp.int32).at[minus_req].add(1)
    offs_m = jnp.cumsum(cnt_m) - cnt_m
    rank_m = jnp.arange(n_minus, dtype=jnp.int32) - offs_m[minus_req]
    slot_m = (cnt_p[minus_req] + rank_m) * r_tot + minus_req

    tab = _LITS * r_tot
    var_flat = (jnp.zeros((tab,), jnp.int32)
                .at[slot_p].set(plus_var).at[slot_m].set(minus_var))
    scale_flat = (jnp.zeros((tab,), jnp.float32)
                  .at[slot_p].set(1.0).at[slot_m].set(-1.0))
    off_flat = jnp.zeros((tab,), jnp.float32).at[slot_m].set(1.0)

    rows_per_w = batch // _NW
    num_vars = preds.shape[1]
    mesh = plsc.VectorSubcoreMesh(core_axis_name="c", subcore_axis_name="s")
    sc_call = functools.partial(
        pl.kernel,
        out_type=jax.ShapeDtypeStruct((_NW, _LANES), jnp.float32),
        mesh=mesh,
        compiler_params=pltpu.CompilerParams(needs_layout_passes=False),
        scratch_types=[
            pltpu.VMEM((tab,), jnp.int32),
            pltpu.VMEM((tab,), jnp.float32),
            pltpu.VMEM((tab,), jnp.float32),
            pltpu.VMEM((_CHUNK * num_vars,), jnp.float32),
            pltpu.VMEM((_LANES,), jnp.float32),
        ],
    )(functools.partial(_body, rows_per_w=rows_per_w, num_vars=num_vars))

    partial = sc_call(preds.reshape(-1), var_flat, scale_flat, off_flat)
    total = jnp.sum(partial)
    denom = jnp.float32(r_tot * batch)
    return jnp.float32(1.0) - total / denom
